# baseline (speedup 1.0000x reference)
import jax
import jax.numpy as jnp
from jax.experimental import pallas as pl
from jax.experimental.pallas import tpu as pltpu

N = 10000
F = 128
ALPHA = 0.3

_BM = 400
_BM_BF = 1000
_BM_SMALL = 1000


def _leaky(x):
    return jnp.where(x >= 0, x, ALPHA * x)


def _bf(x):
    return x.astype(jnp.bfloat16)


def _dot(a, b):
    return jnp.dot(a, b, preferred_element_type=jnp.float32)


def _proj_body(x_ref, w_ref, o_ref):
    o_ref[...] = _bf(_dot(_bf(x_ref[...]), _bf(w_ref[...])))


def _proj(x, w):
    return pl.pallas_call(
        _proj_body,
        grid=(N // _BM_SMALL,),
        in_specs=[pl.BlockSpec((_BM_SMALL, F), lambda i: (i, 0)),
                  pl.BlockSpec((F, F), lambda i: (0, 0))],
        out_specs=pl.BlockSpec((_BM_SMALL, F), lambda i: (i, 0)),
        out_shape=jax.ShapeDtypeStruct((N, F), jnp.bfloat16),
    )(x, w)


def _pass(body, adj, rhs, row_ins, res_ins, out_w, out_dt, bm):
    w = rhs.shape[1]
    in_specs = [pl.BlockSpec((bm, N), lambda i: (i, 0)),
                pl.BlockSpec((N, w), lambda i: (0, 0))]
    for a in row_ins:
        in_specs.append(pl.BlockSpec((bm, a.shape[1]), lambda i: (i, 0)))
    for a in res_ins:
        in_specs.append(pl.BlockSpec(
            tuple(a.shape), lambda i, n=len(a.shape): (0,) * n))
    out_specs = [pl.BlockSpec((bm, ww), lambda i: (i, 0)) for ww in out_w]
    out_shape = [jax.ShapeDtypeStruct((N, ww), dt)
                 for ww, dt in zip(out_w, out_dt)]
    return pl.pallas_call(
        body,
        grid=(N // bm,),
        in_specs=in_specs,
        out_specs=out_specs,
        out_shape=out_shape,
        compiler_params=pltpu.CompilerParams(
            dimension_semantics=("arbitrary",)),
    )(adj, rhs, *row_ins, *res_ins)


def _gcn_tile(a, rhs_ref, b):
    return _leaky(_dot(a, rhs_ref[...]) + b)


def _passA_body(adj_ref, rhs_ref, vfea_ref, b1_ref, w2_ref, w3_ref,
                abf_ref, rB_ref):
    a = _bf(adj_ref[...])
    abf_ref[...] = a
    uh1 = _gcn_tile(a, rhs_ref, b1_ref[...])
    rB_ref[...] = jnp.concatenate(
        [_bf(_dot(_bf(vfea_ref[...]), _bf(w2_ref[...]))),
         _bf(_dot(_bf(uh1), _bf(w3_ref[...])))], axis=1)


def _passB_body(adj_ref, rhs_ref, ufea_ref, b2_ref, b3_ref,
                w4_ref, wll1_ref, wuu_ref, buu_ref,
                rC_ref, u_ref):
    t = _gcn_tile(_bf(adj_ref[...]), rhs_ref,
                  jnp.concatenate([b2_ref[...], b3_ref[...]], axis=1))
    ih1, uh2 = t[:, :F], t[:, F:]
    wuu = _bf(wuu_ref[...])
    u = jnp.maximum(_dot(_bf(uh2), wuu[:F])
                    + _dot(_bf(ufea_ref[...]), wuu[F:])
                    + buu_ref[...], 0.0)
    u_ref[...] = u
    rC_ref[...] = jnp.concatenate(
        [_bf(_dot(_bf(ih1), _bf(w4_ref[...]))),
         _bf(_dot(_bf(u), _bf(wll1_ref[...])))], axis=1)


def _passC_body(adj_ref, rhs_ref, vfea_ref, b4_ref, bll1_ref,
                wll2_ref, w3m_ref, w3s_ref, wiu_ref, biu_ref,
                rD_ref, v_ref):
    t = _gcn_tile(adj_ref[...], rhs_ref,
                  jnp.concatenate([b4_ref[...], bll1_ref[...]], axis=1))
    ih2, uhh = t[:, :F], t[:, F:]
    wiu = _bf(wiu_ref[...])
    v = jnp.maximum(_dot(_bf(ih2), wiu[:F])
                    + _dot(_bf(vfea_ref[...]), wiu[F:])
                    + biu_ref[...], 0.0)
    v_ref[...] = v
    uhh_bf = _bf(uhh)
    rD_ref[...] = jnp.concatenate(
        [_bf(_dot(_bf(v), _bf(wll2_ref[...]))),
         _bf(_dot(uhh_bf, _bf(w3m_ref[...]))),
         _bf(_dot(uhh_bf, _bf(w3s_ref[...])))], axis=1)


def _passD_body(adj_ref, rhs_ref, u_ref, bll2_ref, b3m_ref, b3s_ref,
                w4m_ref, w4s_ref, wum_ref, bum_ref, wus_ref, bus_ref,
                rE_ref, mu_ref, lu_ref):
    t = _gcn_tile(_bf(adj_ref[...]), rhs_ref,
                  jnp.concatenate([bll2_ref[...], b3m_ref[...],
                                   b3s_ref[...]], axis=1))
    ihh, gmu, gsu = t[:, :F], t[:, F:2 * F], t[:, 2 * F:]
    ihh_bf = _bf(ihh)
    rE_ref[...] = jnp.concatenate(
        [_bf(_dot(ihh_bf, _bf(w4m_ref[...]))),
         _bf(_dot(ihh_bf, _bf(w4s_ref[...])))], axis=1)
    ub = _bf(u_ref[...])
    wum = _bf(wum_ref[...])
    wus = _bf(wus_ref[...])
    mu_ref[...] = (_dot(_bf(gmu), wum[:F]) + _dot(ub, wum[F:])
                   + bum_ref[...])
    lu_ref[...] = (_dot(_bf(gsu), wus[:F]) + _dot(ub, wus[F:])
                   + bus_ref[...])


def _passE_body(adj_ref, rhs_ref, v_ref, b4m_ref, b4s_ref,
                wim_ref, bim_ref, wis_ref, bis_ref,
                mi_ref, li_ref):
    t = _gcn_tile(adj_ref[...], rhs_ref,
                  jnp.concatenate([b4m_ref[...], b4s_ref[...]], axis=1))
    gmi, gsi = t[:, :F], t[:, F:]
    vb = _bf(v_ref[...])
    wim = _bf(wim_ref[...])
    wis = _bf(wis_ref[...])
    mi_ref[...] = (_dot(_bf(gmi), wim[:F]) + _dot(vb, wim[F:])
                   + bim_ref[...])
    li_ref[...] = (_dot(_bf(gsi), wis[:F]) + _dot(vb, wis[F:])
                   + bis_ref[...])


def kernel(ufea, vfea, UV_adj, VU_adj, params):
    p = params
    b = {k: v[None, :] for k, v in p.items() if k.endswith('_b')}

    rA = _proj(ufea, p['l0_gc1_W'])

    vu_bf, rB = _pass(
        _passA_body, VU_adj, rA, [vfea],
        [b['l0_gc1_b'], p['l0_gc2_W'], p['l0_gc3_W']],
        [N, 2 * F], [jnp.bfloat16, jnp.bfloat16], _BM)

    rC, u = _pass(
        _passB_body, UV_adj, rB, [ufea],
        [b['l0_gc2_b'], b['l0_gc3_b'],
         p['l0_gc4_W'], p['ll_gc1_W'], p['l0_uu_W'], b['l0_uu_b']],
        [2 * F, F], [jnp.bfloat16, jnp.float32], _BM)

    rD, v = _pass(
        _passC_body, vu_bf, rC, [vfea],
        [b['l0_gc4_b'], b['ll_gc1_b'],
         p['ll_gc2_W'], p['ll_gc3m_W'], p['ll_gc3s_W'],
         p['l0_iu_W'], b['l0_iu_b']],
        [3 * F, F], [jnp.bfloat16, jnp.float32], _BM_BF)

    rE, mean_u, logstd_u = _pass(
        _passD_body, UV_adj, rD, [u],
        [b['ll_gc2_b'], b['ll_gc3m_b'], b['ll_gc3s_b'],
         p['ll_gc4m_W'], p['ll_gc4s_W'],
         p['ll_uum_W'], b['ll_uum_b'], p['ll_uus_W'], b['ll_uus_b']],
        [2 * F, F, F], [jnp.bfloat16, jnp.float32, jnp.float32], _BM)

    mean_i, logstd_i = _pass(
        _passE_body, vu_bf, rE, [v],
        [b['ll_gc4m_b'], b['ll_gc4s_b'],
         p['ll_ium_W'], b['ll_ium_b'], p['ll_ius_W'], b['ll_ius_b']],
        [F, F], [jnp.float32, jnp.float32], _BM_BF)

    return (mean_u, mean_i, mean_u, mean_i, logstd_u, logstd_i)


# speedup vs baseline: 1.0221x; 1.0221x over previous
import jax
import jax.numpy as jnp
from jax.experimental import pallas as pl
from jax.experimental.pallas import tpu as pltpu

N = 10000
F = 128
ALPHA = 0.3

_BM = 400
_BM_BF = 1000
_BM_SMALL = 1000


def _leaky(x):
    return jnp.where(x >= 0, x, ALPHA * x)


def _bf(x):
    return x.astype(jnp.bfloat16)


def _dot(a, b):
    return jnp.dot(a, b, preferred_element_type=jnp.float32)


def _proj_body(x_ref, w_ref, o_ref):
    o_ref[...] = _bf(_dot(_bf(x_ref[...]), w_ref[...]))


def _proj(x, w_bf):
    return pl.pallas_call(
        _proj_body,
        grid=(N // _BM_SMALL,),
        in_specs=[pl.BlockSpec((_BM_SMALL, F), lambda i: (i, 0)),
                  pl.BlockSpec((F, F), lambda i: (0, 0))],
        out_specs=pl.BlockSpec((_BM_SMALL, F), lambda i: (i, 0)),
        out_shape=jax.ShapeDtypeStruct((N, F), jnp.bfloat16),
    )(x, w_bf)


def _pass(body, adj, rhs, bias, row_ins, res_ins, out_w, out_dt, bm):
    w = rhs.shape[1]
    in_specs = [pl.BlockSpec((bm, N), lambda i: (i, 0)),
                pl.BlockSpec((N, w), lambda i: (0, 0)),
                pl.BlockSpec((1, w), lambda i: (0, 0))]
    for a in row_ins:
        in_specs.append(pl.BlockSpec((bm, a.shape[1]), lambda i: (i, 0)))
    for a in res_ins:
        in_specs.append(pl.BlockSpec(
            tuple(a.shape), lambda i, n=len(a.shape): (0,) * n))
    out_specs = [pl.BlockSpec((bm, ww), lambda i: (i, 0)) for ww in out_w]
    out_shape = [jax.ShapeDtypeStruct((N, ww), dt)
                 for ww, dt in zip(out_w, out_dt)]
    return pl.pallas_call(
        body,
        grid=(N // bm,),
        in_specs=in_specs,
        out_specs=out_specs,
        out_shape=out_shape,
        compiler_params=pltpu.CompilerParams(
            dimension_semantics=("arbitrary",)),
    )(adj, rhs, bias, *row_ins, *res_ins)


def _gcn_tile(adj_ref, rhs_ref, b_ref):
    a = adj_ref[...]
    if a.dtype != jnp.bfloat16:
        a = _bf(a)
    return _leaky(_dot(a, rhs_ref[...]) + b_ref[...])


def _passA_body(adj_ref, rhs_ref, b_ref, vfea_ref, w2_ref, w3_ref,
                abf_ref, rB_ref):
    a = _bf(adj_ref[...])
    abf_ref[...] = a
    uh1 = _leaky(_dot(a, rhs_ref[...]) + b_ref[...])
    rB_ref[...] = jnp.concatenate(
        [_bf(_dot(_bf(vfea_ref[...]), w2_ref[...])),
         _bf(_dot(_bf(uh1), w3_ref[...]))], axis=1)


def _passB_body(adj_ref, rhs_ref, b_ref, ufea_ref,
                w4_ref, wll1_ref, wuu1_ref, wuu2_ref, buu_ref,
                rC_ref, u_ref):
    t = _gcn_tile(adj_ref, rhs_ref, b_ref)
    ih1, uh2 = t[:, :F], t[:, F:]
    u = jnp.maximum(_dot(_bf(uh2), wuu1_ref[...])
                    + _dot(_bf(ufea_ref[...]), wuu2_ref[...])
                    + buu_ref[...], 0.0)
    u_ref[...] = u
    rC_ref[...] = jnp.concatenate(
        [_bf(_dot(_bf(ih1), w4_ref[...])),
         _bf(_dot(_bf(u), wll1_ref[...]))], axis=1)


def _passC_body(adj_ref, rhs_ref, b_ref, vfea_ref,
                wll2_ref, w3ms_ref, wiu1_ref, wiu2_ref, biu_ref,
                rD_ref, v_ref):
    t = _gcn_tile(adj_ref, rhs_ref, b_ref)
    ih2, uhh = t[:, :F], t[:, F:]
    v = jnp.maximum(_dot(_bf(ih2), wiu1_ref[...])
                    + _dot(_bf(vfea_ref[...]), wiu2_ref[...])
                    + biu_ref[...], 0.0)
    v_ref[...] = v
    rD_ref[...] = jnp.concatenate(
        [_bf(_dot(_bf(v), wll2_ref[...])),
         _bf(_dot(_bf(uhh), w3ms_ref[...]))], axis=1)


def _passD_body(adj_ref, rhs_ref, b_ref, u_ref,
                w4ms_ref, wum1_ref, wum2_ref, bum_ref,
                wus1_ref, wus2_ref, bus_ref,
                rE_ref, mu_ref, lu_ref):
    t = _gcn_tile(adj_ref, rhs_ref, b_ref)
    ihh, gmu, gsu = t[:, :F], t[:, F:2 * F], t[:, 2 * F:]
    rE_ref[...] = _bf(_dot(_bf(ihh), w4ms_ref[...]))
    ub = _bf(u_ref[...])
    mu_ref[...] = (_dot(_bf(gmu), wum1_ref[...]) + _dot(ub, wum2_ref[...])
                   + bum_ref[...])
    lu_ref[...] = (_dot(_bf(gsu), wus1_ref[...]) + _dot(ub, wus2_ref[...])
                   + bus_ref[...])


def _passE_body(adj_ref, rhs_ref, b_ref, v_ref,
                wim1_ref, wim2_ref, bim_ref,
                wis1_ref, wis2_ref, bis_ref,
                mi_ref, li_ref):
    t = _gcn_tile(adj_ref, rhs_ref, b_ref)
    gmi, gsi = t[:, :F], t[:, F:]
    vb = _bf(v_ref[...])
    mi_ref[...] = (_dot(_bf(gmi), wim1_ref[...]) + _dot(vb, wim2_ref[...])
                   + bim_ref[...])
    li_ref[...] = (_dot(_bf(gsi), wis1_ref[...]) + _dot(vb, wis2_ref[...])
                   + bis_ref[...])


def kernel(ufea, vfea, UV_adj, VU_adj, params):
    p = params

    def wcat(*names):
        return _bf(jnp.concatenate([p[n] for n in names], axis=1))

    def bcat(*names):
        return jnp.concatenate([p[n] for n in names])[None, :]

    rA = _proj(ufea, _bf(p['l0_gc1_W']))

    vu_bf, rB = _pass(
        _passA_body, VU_adj, rA, p['l0_gc1_b'][None, :],
        [vfea], [_bf(p['l0_gc2_W']), _bf(p['l0_gc3_W'])],
        [N, 2 * F], [jnp.bfloat16, jnp.bfloat16], _BM)

    rC, u = _pass(
        _passB_body, UV_adj, rB, bcat('l0_gc2_b', 'l0_gc3_b'),
        [ufea],
        [_bf(p['l0_gc4_W']), _bf(p['ll_gc1_W']),
         _bf(p['l0_uu_W'][:F]), _bf(p['l0_uu_W'][F:]), p['l0_uu_b'][None, :]],
        [2 * F, F], [jnp.bfloat16, jnp.float32], _BM)

    rD, v = _pass(
        _passC_body, vu_bf, rC, bcat('l0_gc4_b', 'll_gc1_b'),
        [vfea],
        [_bf(p['ll_gc2_W']), wcat('ll_gc3m_W', 'll_gc3s_W'),
         _bf(p['l0_iu_W'][:F]), _bf(p['l0_iu_W'][F:]), p['l0_iu_b'][None, :]],
        [3 * F, F], [jnp.bfloat16, jnp.float32], _BM_BF)

    rE, mean_u, logstd_u = _pass(
        _passD_body, UV_adj, rD, bcat('ll_gc2_b', 'll_gc3m_b', 'll_gc3s_b'),
        [u],
        [wcat('ll_gc4m_W', 'll_gc4s_W'),
         _bf(p['ll_uum_W'][:F]), _bf(p['ll_uum_W'][F:]), p['ll_uum_b'][None, :],
         _bf(p['ll_uus_W'][:F]), _bf(p['ll_uus_W'][F:]), p['ll_uus_b'][None, :]],
        [2 * F, F, F], [jnp.bfloat16, jnp.float32, jnp.float32], _BM)

    mean_i, logstd_i = _pass(
        _passE_body, vu_bf, rE, bcat('ll_gc4m_b', 'll_gc4s_b'),
        [v],
        [_bf(p['ll_ium_W'][:F]), _bf(p['ll_ium_W'][F:]), p['ll_ium_b'][None, :],
         _bf(p['ll_ius_W'][:F]), _bf(p['ll_ius_W'][F:]), p['ll_ius_b'][None, :]],
        [F, F], [jnp.float32, jnp.float32], _BM_BF)

    return (mean_u, mean_i, mean_u, mean_i, logstd_u, logstd_i)


# speedup vs baseline: 1.0221x; 1.0001x over previous
import jax
import jax.numpy as jnp
from jax.experimental import pallas as pl
from jax.experimental.pallas import tpu as pltpu

N = 10000
F = 128
ALPHA = 0.3

_BM = 400
_BM_BF = 1000
_BM_SMALL = 1000


def _leaky(x):
    return jnp.where(x >= 0, x, ALPHA * x)


def _bf(x):
    return x.astype(jnp.bfloat16)


def _dot(a, b):
    return jnp.dot(a, b, preferred_element_type=jnp.float32)


def _proj_body(x_ref, w_ref, o_ref):
    o_ref[...] = _bf(_dot(_bf(x_ref[...]), w_ref[...]))


def _proj(x, w_bf):
    return pl.pallas_call(
        _proj_body,
        grid=(N // _BM_SMALL,),
        in_specs=[pl.BlockSpec((_BM_SMALL, F), lambda i: (i, 0)),
                  pl.BlockSpec((F, F), lambda i: (0, 0))],
        out_specs=pl.BlockSpec((_BM_SMALL, F), lambda i: (i, 0)),
        out_shape=jax.ShapeDtypeStruct((N, F), jnp.bfloat16),
    )(x, w_bf)


def _pass(body, adj, rhs, bias, row_ins, res_ins, out_w, out_dt, bm):
    w = rhs.shape[1]
    in_specs = [pl.BlockSpec((bm, N), lambda i: (i, 0)),
                pl.BlockSpec((N, w), lambda i: (0, 0)),
                pl.BlockSpec((1, w), lambda i: (0, 0))]
    for a in row_ins:
        in_specs.append(pl.BlockSpec((bm, a.shape[1]), lambda i: (i, 0)))
    for a in res_ins:
        in_specs.append(pl.BlockSpec(
            tuple(a.shape), lambda i, n=len(a.shape): (0,) * n))
    out_specs = [pl.BlockSpec((bm, ww), lambda i: (i, 0)) for ww in out_w]
    out_shape = [jax.ShapeDtypeStruct((N, ww), dt)
                 for ww, dt in zip(out_w, out_dt)]
    return pl.pallas_call(
        body,
        grid=(N // bm,),
        in_specs=in_specs,
        out_specs=out_specs,
        out_shape=out_shape,
        compiler_params=pltpu.CompilerParams(
            dimension_semantics=("parallel",)),
    )(adj, rhs, bias, *row_ins, *res_ins)


def _gcn_tile(adj_ref, rhs_ref, b_ref):
    a = adj_ref[...]
    if a.dtype != jnp.bfloat16:
        a = _bf(a)
    return _leaky(_dot(a, rhs_ref[...]) + b_ref[...])


def _passA_body(adj_ref, rhs_ref, b_ref, vfea_ref, w2_ref, w3_ref,
                abf_ref, rB_ref):
    a = _bf(adj_ref[...])
    abf_ref[...] = a
    uh1 = _leaky(_dot(a, rhs_ref[...]) + b_ref[...])
    rB_ref[...] = jnp.concatenate(
        [_bf(_dot(_bf(vfea_ref[...]), w2_ref[...])),
         _bf(_dot(_bf(uh1), w3_ref[...]))], axis=1)


def _passB_body(adj_ref, rhs_ref, b_ref, ufea_ref,
                w4_ref, wll1_ref, wuu1_ref, wuu2_ref, buu_ref,
                rC_ref, u_ref):
    t = _gcn_tile(adj_ref, rhs_ref, b_ref)
    ih1, uh2 = t[:, :F], t[:, F:]
    u = jnp.maximum(_dot(_bf(uh2), wuu1_ref[...])
                    + _dot(_bf(ufea_ref[...]), wuu2_ref[...])
                    + buu_ref[...], 0.0)
    u_ref[...] = u
    rC_ref[...] = jnp.concatenate(
        [_bf(_dot(_bf(ih1), w4_ref[...])),
         _bf(_dot(_bf(u), wll1_ref[...]))], axis=1)


def _passC_body(adj_ref, rhs_ref, b_ref, vfea_ref,
                wll2_ref, w3ms_ref, wiu1_ref, wiu2_ref, biu_ref,
                rD_ref, v_ref):
    t = _gcn_tile(adj_ref, rhs_ref, b_ref)
    ih2, uhh = t[:, :F], t[:, F:]
    v = jnp.maximum(_dot(_bf(ih2), wiu1_ref[...])
                    + _dot(_bf(vfea_ref[...]), wiu2_ref[...])
                    + biu_ref[...], 0.0)
    v_ref[...] = v
    rD_ref[...] = jnp.concatenate(
        [_bf(_dot(_bf(v), wll2_ref[...])),
         _bf(_dot(_bf(uhh), w3ms_ref[...]))], axis=1)


def _passD_body(adj_ref, rhs_ref, b_ref, u_ref,
                w4ms_ref, wum1_ref, wum2_ref, bum_ref,
                wus1_ref, wus2_ref, bus_ref,
                rE_ref, mu_ref, lu_ref):
    t = _gcn_tile(adj_ref, rhs_ref, b_ref)
    ihh, gmu, gsu = t[:, :F], t[:, F:2 * F], t[:, 2 * F:]
    rE_ref[...] = _bf(_dot(_bf(ihh), w4ms_ref[...]))
    ub = _bf(u_ref[...])
    mu_ref[...] = (_dot(_bf(gmu), wum1_ref[...]) + _dot(ub, wum2_ref[...])
                   + bum_ref[...])
    lu_ref[...] = (_dot(_bf(gsu), wus1_ref[...]) + _dot(ub, wus2_ref[...])
                   + bus_ref[...])


def _passE_body(adj_ref, rhs_ref, b_ref, v_ref,
                wim1_ref, wim2_ref, bim_ref,
                wis1_ref, wis2_ref, bis_ref,
                mi_ref, li_ref):
    t = _gcn_tile(adj_ref, rhs_ref, b_ref)
    gmi, gsi = t[:, :F], t[:, F:]
    vb = _bf(v_ref[...])
    mi_ref[...] = (_dot(_bf(gmi), wim1_ref[...]) + _dot(vb, wim2_ref[...])
                   + bim_ref[...])
    li_ref[...] = (_dot(_bf(gsi), wis1_ref[...]) + _dot(vb, wis2_ref[...])
                   + bis_ref[...])


def kernel(ufea, vfea, UV_adj, VU_adj, params):
    p = params

    def wcat(*names):
        return _bf(jnp.concatenate([p[n] for n in names], axis=1))

    def bcat(*names):
        return jnp.concatenate([p[n] for n in names])[None, :]

    rA = _proj(ufea, _bf(p['l0_gc1_W']))

    vu_bf, rB = _pass(
        _passA_body, VU_adj, rA, p['l0_gc1_b'][None, :],
        [vfea], [_bf(p['l0_gc2_W']), _bf(p['l0_gc3_W'])],
        [N, 2 * F], [jnp.bfloat16, jnp.bfloat16], _BM)

    rC, u = _pass(
        _passB_body, UV_adj, rB, bcat('l0_gc2_b', 'l0_gc3_b'),
        [ufea],
        [_bf(p['l0_gc4_W']), _bf(p['ll_gc1_W']),
         _bf(p['l0_uu_W'][:F]), _bf(p['l0_uu_W'][F:]), p['l0_uu_b'][None, :]],
        [2 * F, F], [jnp.bfloat16, jnp.float32], _BM)

    rD, v = _pass(
        _passC_body, vu_bf, rC, bcat('l0_gc4_b', 'll_gc1_b'),
        [vfea],
        [_bf(p['ll_gc2_W']), wcat('ll_gc3m_W', 'll_gc3s_W'),
         _bf(p['l0_iu_W'][:F]), _bf(p['l0_iu_W'][F:]), p['l0_iu_b'][None, :]],
        [3 * F, F], [jnp.bfloat16, jnp.float32], _BM_BF)

    rE, mean_u, logstd_u = _pass(
        _passD_body, UV_adj, rD, bcat('ll_gc2_b', 'll_gc3m_b', 'll_gc3s_b'),
        [u],
        [wcat('ll_gc4m_W', 'll_gc4s_W'),
         _bf(p['ll_uum_W'][:F]), _bf(p['ll_uum_W'][F:]), p['ll_uum_b'][None, :],
         _bf(p['ll_uus_W'][:F]), _bf(p['ll_uus_W'][F:]), p['ll_uus_b'][None, :]],
        [2 * F, F, F], [jnp.bfloat16, jnp.float32, jnp.float32], _BM)

    mean_i, logstd_i = _pass(
        _passE_body, vu_bf, rE, bcat('ll_gc4m_b', 'll_gc4s_b'),
        [v],
        [_bf(p['ll_ium_W'][:F]), _bf(p['ll_ium_W'][F:]), p['ll_ium_b'][None, :],
         _bf(p['ll_ius_W'][:F]), _bf(p['ll_ius_W'][F:]), p['ll_ius_b'][None, :]],
        [F, F], [jnp.float32, jnp.float32], _BM_BF)

    return (mean_u, mean_i, mean_u, mean_i, logstd_u, logstd_i)
